# Initial kernel scaffold; baseline (speedup 1.0000x reference)
import jax
import jax.numpy as jnp
from jax.experimental import pallas as pl


def kernel(x, params):
    raise NotImplementedError("write your pallas kernel here")


# speedup vs baseline: 6.2440x; 6.2440x over previous
import functools

import jax
import jax.numpy as jnp
from jax import lax
from jax.experimental import pallas as pl
from jax.experimental.pallas import tpu as pltpu


def _pick_block(n, candidates):
    for c in candidates:
        if n % c == 0:
            return c
    return n


def _leaky(x):
    return jnp.where(x >= 0, x, 0.2 * x)


def _rownorm(h):
    n2 = jnp.sum(h * h, axis=1, keepdims=True)
    inv = lax.rsqrt(jnp.where(n2 > 0, n2, 1.0))
    return jnp.where(n2 > 0, h * inv, 0.0)


def _gru_body(x_ref, wi0, wh0, bi0, bh0, wi1, wh1, bi1, bh1,
              h_out, hn_out, h1_s, h2_s):
    t = pl.program_id(1)
    T = pl.num_programs(1)
    H = wh0.shape[0]

    @pl.when(t == 0)
    def _():
        h1_s[...] = jnp.zeros_like(h1_s)
        h2_s[...] = jnp.zeros_like(h2_s)

    xt = x_ref[0]
    h1 = h1_s[...]
    h2 = h2_s[...]

    gi = jnp.dot(xt, wi0[...]) + bi0[...]
    gh = jnp.dot(h1, wh0[...]) + bh0[...]
    r = jax.nn.sigmoid(gi[:, :H] + gh[:, :H])
    z = jax.nn.sigmoid(gi[:, H:2 * H] + gh[:, H:2 * H])
    nn_ = jnp.tanh(gi[:, 2 * H:] + r * gh[:, 2 * H:])
    h1n = (1.0 - z) * nn_ + z * h1
    h1_s[...] = h1n

    gi2 = jnp.dot(h1n, wi1[...]) + bi1[...]
    gh2 = jnp.dot(h2, wh1[...]) + bh1[...]
    r2 = jax.nn.sigmoid(gi2[:, :H] + gh2[:, :H])
    z2 = jax.nn.sigmoid(gi2[:, H:2 * H] + gh2[:, H:2 * H])
    nn2 = jnp.tanh(gi2[:, 2 * H:] + r2 * gh2[:, 2 * H:])
    h2n = (1.0 - z2) * nn2 + z2 * h2
    h2_s[...] = h2n

    @pl.when(t == T - 1)
    def _():
        h_out[...] = h2n
        hn_out[...] = _rownorm(h2n)


def _run_gru(x, p):
    N, T, D = x.shape
    H = p['W_hh0'].shape[1]
    Bg = _pick_block(N, [1000, 400, 200, 80, 40, 16, 8])
    G = N // Bg
    xt = jnp.swapaxes(x, 0, 1)
    f32 = jnp.float32

    wspec = pl.BlockSpec(None, lambda g, t: (0, 0))
    bspec = pl.BlockSpec(None, lambda g, t: (0, 0))
    h_out, hn_out = pl.pallas_call(
        _gru_body,
        grid=(G, T),
        in_specs=[
            pl.BlockSpec((1, Bg, D), lambda g, t: (t, g, 0)),
            wspec, wspec, bspec, bspec, wspec, wspec, bspec, bspec,
        ],
        out_specs=[
            pl.BlockSpec((Bg, H), lambda g, t: (g, 0)),
            pl.BlockSpec((Bg, H), lambda g, t: (g, 0)),
        ],
        out_shape=[
            jax.ShapeDtypeStruct((N, H), f32),
            jax.ShapeDtypeStruct((N, H), f32),
        ],
        scratch_shapes=[
            pltpu.VMEM((Bg, H), f32),
            pltpu.VMEM((Bg, H), f32),
        ],
    )(
        xt,
        p['W_ih0'].T, p['W_hh0'].T, p['b_ih0'].reshape(1, -1), p['b_hh0'].reshape(1, -1),
        p['W_ih1'].T, p['W_hh1'].T, p['b_ih1'].reshape(1, -1), p['b_hh1'].reshape(1, -1),
    )
    return h_out, hn_out


def _topk_body(hn_blk, hn_full, h_blk, pre_ref, cs_ref, *, K, B):
    i = pl.program_id(0)
    N = hn_full.shape[0]

    hnb = hn_blk[...]
    S = lax.dot_general(hnb, hn_full[...], (((1,), (1,)), ((), ())))
    rows = i * B + lax.broadcasted_iota(jnp.int32, (B, N), 0)
    cols = lax.broadcasted_iota(jnp.int32, (B, N), 1)
    S = jnp.where(rows == cols, 0.0, S)
    A = jnp.abs(S)

    def body(_, carry):
        v, _t = carry
        m = jnp.max(v, axis=1, keepdims=True)
        return jnp.where(v >= m, -1.0, v), m

    _, thr = lax.fori_loop(0, K, body, (A, jnp.zeros((B, 1), jnp.float32)))
    TS = jnp.where(A >= thr, S, 0.0)

    @pl.when(i == 0)
    def _():
        pre_ref[...] = jnp.zeros_like(pre_ref)
        cs_ref[...] = jnp.zeros_like(cs_ref)

    pre_ref[...] += lax.dot_general(TS, h_blk[...], (((0,), (0,)), ((), ())))
    cs_ref[...] += lax.dot_general(TS, jnp.ones((B, 1), jnp.float32),
                                   (((0,), (0,)), ((), ())))


def _run_topk_accum(hn, h, K):
    N, H = h.shape
    B = _pick_block(N, [200, 80, 40, 16, 8])
    f32 = jnp.float32
    pre, cs = pl.pallas_call(
        functools.partial(_topk_body, K=K, B=B),
        grid=(N // B,),
        in_specs=[
            pl.BlockSpec((B, H), lambda i: (i, 0)),
            pl.BlockSpec(None, lambda i: (0, 0)),
            pl.BlockSpec((B, H), lambda i: (i, 0)),
        ],
        out_specs=[
            pl.BlockSpec(None, lambda i: (0, 0)),
            pl.BlockSpec(None, lambda i: (0, 0)),
        ],
        out_shape=[
            jax.ShapeDtypeStruct((N, H), f32),
            jax.ShapeDtypeStruct((N, 1), f32),
        ],
    )(hn, hn, h)
    return pre, cs


def _concept_body(pre_ref, cs_ref, hn_ref, h_ref, wc, bc, concept_ref, cn_ref):
    hn = hn_ref[...]
    n2 = jnp.sum(hn * hn, axis=1, keepdims=True)
    d = jnp.where((cs_ref[...] != 0) & (n2 > 0.5), 1.0, 0.0)
    pre_c = pre_ref[...] + d * h_ref[...]
    valid = jnp.where(jnp.sum(pre_c, axis=1, keepdims=True) != 0, 1.0, 0.0)
    concept = _leaky(jnp.dot(pre_c, wc[...]) + bc[...]) * valid
    concept_ref[...] = concept
    cn_ref[...] = _rownorm(concept)


def _run_concept(pre, cs, hn, h, p):
    N, H = h.shape
    f32 = jnp.float32
    spec = pl.BlockSpec(None, lambda: (0, 0))
    return pl.pallas_call(
        _concept_body,
        in_specs=[spec] * 6,
        out_specs=[spec, spec],
        out_shape=[
            jax.ShapeDtypeStruct((N, H), f32),
            jax.ShapeDtypeStruct((N, H), f32),
        ],
    )(pre, cs, hn, h, p['W_c'].T, p['b_c'].reshape(1, -1))


def _softstat_body(hn_blk, cn_full, m_ref, s_ref):
    i = pl.program_id(0)
    C = lax.dot_general(hn_blk[...], cn_full[...], (((1,), (1,)), ((), ())))
    mb = jnp.max(C, axis=0, keepdims=True)

    @pl.when(i == 0)
    def _():
        m_ref[...] = jnp.full_like(m_ref, -jnp.inf)
        s_ref[...] = jnp.zeros_like(s_ref)

    m_old = m_ref[...]
    m_new = jnp.maximum(m_old, mb)
    s_ref[...] = (s_ref[...] * jnp.exp(m_old - m_new)
                  + jnp.sum(jnp.exp(C - m_new), axis=0, keepdims=True))
    m_ref[...] = m_new


def _run_softstat(hn, cn):
    N, H = hn.shape
    B = _pick_block(N, [400, 200, 80, 40, 16, 8])
    f32 = jnp.float32
    return pl.pallas_call(
        _softstat_body,
        grid=(N // B,),
        in_specs=[
            pl.BlockSpec((B, H), lambda i: (i, 0)),
            pl.BlockSpec(None, lambda i: (0, 0)),
        ],
        out_specs=[
            pl.BlockSpec(None, lambda i: (0, 0)),
            pl.BlockSpec(None, lambda i: (0, 0)),
        ],
        out_shape=[
            jax.ShapeDtypeStruct((1, N), f32),
            jax.ShapeDtypeStruct((1, N), f32),
        ],
    )(hn, cn)


def _head_body(hn_blk, cn_full, concept_full, h_blk, m_ref, s_ref,
               whs, bhs, wback, bback, wfore, bfore, windi, bindi,
               wout, bout, pred_ref):
    C = lax.dot_general(hn_blk[...], cn_full[...], (((1,), (1,)), ((), ())))
    att = jnp.exp(C - m_ref[...]) / s_ref[...]
    hsh = jnp.dot(att, concept_full[...])
    hs = _leaky(jnp.dot(hsh, whs[...]) + bhs[...])
    hb = _leaky(jnp.dot(hs, wback[...]) + bback[...])
    ofs = _leaky(jnp.dot(hs, wfore[...]) + bfore[...])
    indi = h_blk[...] - hb
    oin = _leaky(jnp.dot(indi, windi[...]) + bindi[...])
    pred_ref[...] = jnp.dot(ofs + oin, wout[...]) + bout[...]


def _run_head(hn, cn, concept, h, m, s, p):
    N, H = hn.shape
    B = _pick_block(N, [400, 200, 80, 40, 16, 8])
    f32 = jnp.float32
    wspec = pl.BlockSpec(None, lambda i: (0, 0))
    pred = pl.pallas_call(
        _head_body,
        grid=(N // B,),
        in_specs=[
            pl.BlockSpec((B, H), lambda i: (i, 0)),
            wspec,
            wspec,
            pl.BlockSpec((B, H), lambda i: (i, 0)),
            wspec, wspec,
            wspec, wspec, wspec, wspec, wspec,
            wspec, wspec, wspec, wspec, wspec,
        ],
        out_specs=pl.BlockSpec((B, 1), lambda i: (i, 0)),
        out_shape=jax.ShapeDtypeStruct((N, 1), f32),
    )(
        hn, cn, concept, h, m, s,
        p['W_hs'].T, p['b_hs'].reshape(1, -1),
        p['W_back'].T, p['b_back'].reshape(1, -1),
        p['W_fore'].T, p['b_fore'].reshape(1, -1),
        p['W_indi'].T, p['b_indi'].reshape(1, -1),
        p['W_out'].T, p['b_out'].reshape(1, -1),
    )
    return pred


K_TOP = 20


def kernel(x, params):
    h, hn = _run_gru(x, params)
    pre, cs = _run_topk_accum(hn, h, K_TOP)
    concept, cn = _run_concept(pre, cs, hn, h, params)
    m, s = _run_softstat(hn, cn)
    pred = _run_head(hn, cn, concept, h, m, s, params)
    return pred[:, 0]


# speedup vs baseline: 7.0845x; 1.1346x over previous
import functools

import jax
import jax.numpy as jnp
from jax import lax
from jax.experimental import pallas as pl
from jax.experimental.pallas import tpu as pltpu


def _pick_block(n, candidates):
    for c in candidates:
        if n % c == 0:
            return c
    return n


def _leaky(x):
    return jnp.where(x >= 0, x, 0.2 * x)


def _rownorm(h):
    n2 = jnp.sum(h * h, axis=1, keepdims=True)
    inv = lax.rsqrt(jnp.where(n2 > 0, n2, 1.0))
    return jnp.where(n2 > 0, h * inv, 0.0)


def _gru_step(xt, h, wrz, brz, wni, bni, wnh, bnh, H):
    cat = jnp.concatenate([xt, h], axis=1)
    rz = jax.nn.sigmoid(jnp.dot(cat, wrz) + brz)
    r = rz[:, :H]
    z = rz[:, H:]
    gin = jnp.dot(xt, wni) + bni
    ghn = jnp.dot(h, wnh) + bnh
    nn_ = jnp.tanh(gin + r * ghn)
    return (1.0 - z) * nn_ + z * h


def _gru_body(x_ref, wrz0, brz0, wni0, bni0, wnh0, bnh0,
              wrz1, brz1, wni1, bni1, wnh1, bnh1,
              h_out, hn_out, h1_s, h2_s):
    t = pl.program_id(1)
    T = pl.num_programs(1)
    H = wnh0.shape[0]

    @pl.when(t == 0)
    def _():
        h1_s[...] = jnp.zeros_like(h1_s)
        h2_s[...] = jnp.zeros_like(h2_s)

    xt = x_ref[0]
    h1n = _gru_step(xt, h1_s[...], wrz0[...], brz0[...], wni0[...],
                    bni0[...], wnh0[...], bnh0[...], H)
    h1_s[...] = h1n
    h2n = _gru_step(h1n, h2_s[...], wrz1[...], brz1[...], wni1[...],
                    bni1[...], wnh1[...], bnh1[...], H)
    h2_s[...] = h2n

    @pl.when(t == T - 1)
    def _():
        h_out[...] = h2n
        hn_out[...] = _rownorm(h2n)


def _run_gru(x, p):
    N, T, D = x.shape
    H = p['W_hh0'].shape[1]
    Bg = _pick_block(N, [2000, 1000, 400, 200, 80, 40, 16, 8])
    G = N // Bg
    xt = jnp.swapaxes(x, 0, 1)
    f32 = jnp.float32

    def gate_pack(wi, wh, bi, bh):
        wrz = jnp.concatenate([wi.T[:, :2 * H], wh.T[:, :2 * H]], axis=0)
        brz = (bi[:2 * H] + bh[:2 * H]).reshape(1, -1)
        return (wrz, brz, wi.T[:, 2 * H:], bi[2 * H:].reshape(1, -1),
                wh.T[:, 2 * H:], bh[2 * H:].reshape(1, -1))

    l0 = gate_pack(p['W_ih0'], p['W_hh0'], p['b_ih0'], p['b_hh0'])
    l1 = gate_pack(p['W_ih1'], p['W_hh1'], p['b_ih1'], p['b_hh1'])

    wspec = pl.BlockSpec(None, lambda g, t: (0, 0))
    h_out, hn_out = pl.pallas_call(
        _gru_body,
        grid=(G, T),
        in_specs=[
            pl.BlockSpec((1, Bg, D), lambda g, t: (t, g, 0)),
        ] + [wspec] * 12,
        out_specs=[
            pl.BlockSpec((Bg, H), lambda g, t: (g, 0)),
            pl.BlockSpec((Bg, H), lambda g, t: (g, 0)),
        ],
        out_shape=[
            jax.ShapeDtypeStruct((N, H), f32),
            jax.ShapeDtypeStruct((N, H), f32),
        ],
        scratch_shapes=[
            pltpu.VMEM((Bg, H), f32),
            pltpu.VMEM((Bg, H), f32),
        ],
    )(xt, *l0, *l1)
    return h_out, hn_out


def _topk_body(hn_blk, hn_full, h_blk, pre_ref, cs_ref, *, K, B):
    i = pl.program_id(0)
    N = hn_full.shape[0]

    hnb = hn_blk[...]
    S = lax.dot_general(hnb, hn_full[...], (((1,), (1,)), ((), ())))
    rows = i * B + lax.broadcasted_iota(jnp.int32, (B, N), 0)
    cols = lax.broadcasted_iota(jnp.int32, (B, N), 1)
    S = jnp.where(rows == cols, 0.0, S)
    A = jnp.abs(S)

    def body(_, carry):
        v, _t = carry
        m = jnp.max(v, axis=1, keepdims=True)
        return jnp.where(v >= m, -1.0, v), m

    _, thr = lax.fori_loop(0, K, body, (A, jnp.zeros((B, 1), jnp.float32)))
    TS = jnp.where(A >= thr, S, 0.0)

    @pl.when(i == 0)
    def _():
        pre_ref[...] = jnp.zeros_like(pre_ref)
        cs_ref[...] = jnp.zeros_like(cs_ref)

    pre_ref[...] += lax.dot_general(TS, h_blk[...], (((0,), (0,)), ((), ())))
    cs_ref[...] += lax.dot_general(TS, jnp.ones((B, 1), jnp.float32),
                                   (((0,), (0,)), ((), ())))


def _run_topk_accum(hn, h, K):
    N, H = h.shape
    B = _pick_block(N, [200, 80, 40, 16, 8])
    f32 = jnp.float32
    pre, cs = pl.pallas_call(
        functools.partial(_topk_body, K=K, B=B),
        grid=(N // B,),
        in_specs=[
            pl.BlockSpec((B, H), lambda i: (i, 0)),
            pl.BlockSpec(None, lambda i: (0, 0)),
            pl.BlockSpec((B, H), lambda i: (i, 0)),
        ],
        out_specs=[
            pl.BlockSpec(None, lambda i: (0, 0)),
            pl.BlockSpec(None, lambda i: (0, 0)),
        ],
        out_shape=[
            jax.ShapeDtypeStruct((N, H), f32),
            jax.ShapeDtypeStruct((N, 1), f32),
        ],
    )(hn, hn, h)
    return pre, cs


def _concept_body(pre_ref, cs_ref, hn_ref, h_ref, wc, bc, concept_ref, cn_ref):
    hn = hn_ref[...]
    n2 = jnp.sum(hn * hn, axis=1, keepdims=True)
    d = jnp.where((cs_ref[...] != 0) & (n2 > 0.5), 1.0, 0.0)
    pre_c = pre_ref[...] + d * h_ref[...]
    valid = jnp.where(jnp.sum(pre_c, axis=1, keepdims=True) != 0, 1.0, 0.0)
    concept = _leaky(jnp.dot(pre_c, wc[...]) + bc[...]) * valid
    concept_ref[...] = concept
    cn_ref[...] = _rownorm(concept)


def _run_concept(pre, cs, hn, h, p):
    N, H = h.shape
    f32 = jnp.float32
    spec = pl.BlockSpec(None, lambda: (0, 0))
    return pl.pallas_call(
        _concept_body,
        in_specs=[spec] * 6,
        out_specs=[spec, spec],
        out_shape=[
            jax.ShapeDtypeStruct((N, H), f32),
            jax.ShapeDtypeStruct((N, H), f32),
        ],
    )(pre, cs, hn, h, p['W_c'].T, p['b_c'].reshape(1, -1))


def _softstat_body(hn_blk, cn_full, m_ref, s_ref):
    i = pl.program_id(0)
    C = lax.dot_general(hn_blk[...], cn_full[...], (((1,), (1,)), ((), ())))
    mb = jnp.max(C, axis=0, keepdims=True)

    @pl.when(i == 0)
    def _():
        m_ref[...] = jnp.full_like(m_ref, -jnp.inf)
        s_ref[...] = jnp.zeros_like(s_ref)

    m_old = m_ref[...]
    m_new = jnp.maximum(m_old, mb)
    s_ref[...] = (s_ref[...] * jnp.exp(m_old - m_new)
                  + jnp.sum(jnp.exp(C - m_new), axis=0, keepdims=True))
    m_ref[...] = m_new


def _run_softstat(hn, cn):
    N, H = hn.shape
    B = _pick_block(N, [400, 200, 80, 40, 16, 8])
    f32 = jnp.float32
    return pl.pallas_call(
        _softstat_body,
        grid=(N // B,),
        in_specs=[
            pl.BlockSpec((B, H), lambda i: (i, 0)),
            pl.BlockSpec(None, lambda i: (0, 0)),
        ],
        out_specs=[
            pl.BlockSpec(None, lambda i: (0, 0)),
            pl.BlockSpec(None, lambda i: (0, 0)),
        ],
        out_shape=[
            jax.ShapeDtypeStruct((1, N), f32),
            jax.ShapeDtypeStruct((1, N), f32),
        ],
    )(hn, cn)


def _head_body(hn_blk, cn_full, concept_full, h_blk, m_ref, s_ref,
               whs, bhs, wback, bback, wfore, bfore, windi, bindi,
               wout, bout, pred_ref):
    C = lax.dot_general(hn_blk[...], cn_full[...], (((1,), (1,)), ((), ())))
    att = jnp.exp(C - m_ref[...]) / s_ref[...]
    hsh = jnp.dot(att, concept_full[...])
    hs = _leaky(jnp.dot(hsh, whs[...]) + bhs[...])
    hb = _leaky(jnp.dot(hs, wback[...]) + bback[...])
    ofs = _leaky(jnp.dot(hs, wfore[...]) + bfore[...])
    indi = h_blk[...] - hb
    oin = _leaky(jnp.dot(indi, windi[...]) + bindi[...])
    pred_ref[...] = jnp.dot(ofs + oin, wout[...]) + bout[...]


def _run_head(hn, cn, concept, h, m, s, p):
    N, H = hn.shape
    B = _pick_block(N, [400, 200, 80, 40, 16, 8])
    f32 = jnp.float32
    wspec = pl.BlockSpec(None, lambda i: (0, 0))
    pred = pl.pallas_call(
        _head_body,
        grid=(N // B,),
        in_specs=[
            pl.BlockSpec((B, H), lambda i: (i, 0)),
            wspec,
            wspec,
            pl.BlockSpec((B, H), lambda i: (i, 0)),
            wspec, wspec,
            wspec, wspec, wspec, wspec, wspec,
            wspec, wspec, wspec, wspec, wspec,
        ],
        out_specs=pl.BlockSpec((B, 1), lambda i: (i, 0)),
        out_shape=jax.ShapeDtypeStruct((N, 1), f32),
    )(
        hn, cn, concept, h, m, s,
        p['W_hs'].T, p['b_hs'].reshape(1, -1),
        p['W_back'].T, p['b_back'].reshape(1, -1),
        p['W_fore'].T, p['b_fore'].reshape(1, -1),
        p['W_indi'].T, p['b_indi'].reshape(1, -1),
        p['W_out'].T, p['b_out'].reshape(1, -1),
    )
    return pred


K_TOP = 20


def kernel(x, params):
    h, hn = _run_gru(x, params)
    pre, cs = _run_topk_accum(hn, h, K_TOP)
    concept, cn = _run_concept(pre, cs, hn, h, params)
    m, s = _run_softstat(hn, cn)
    pred = _run_head(hn, cn, concept, h, m, s, params)
    return pred[:, 0]


# speedup vs baseline: 7.3104x; 1.0319x over previous
import functools

import jax
import jax.numpy as jnp
from jax import lax
from jax.experimental import pallas as pl
from jax.experimental.pallas import tpu as pltpu


def _pick_block(n, candidates):
    for c in candidates:
        if n % c == 0:
            return c
    return n


def _leaky(x):
    return jnp.where(x >= 0, x, 0.2 * x)


def _rownorm(h):
    n2 = jnp.sum(h * h, axis=1, keepdims=True)
    inv = lax.rsqrt(jnp.where(n2 > 0, n2, 1.0))
    return jnp.where(n2 > 0, h * inv, 0.0)


def _gru_step(xt, h, wrz, brz, wni, bni, wnh, bnh, H):
    cat = jnp.concatenate([xt, h], axis=1)
    rz = jax.nn.sigmoid(jnp.dot(cat, wrz) + brz)
    r = rz[:, :H]
    z = rz[:, H:]
    gin = jnp.dot(xt, wni) + bni
    ghn = jnp.dot(h, wnh) + bnh
    nn_ = jnp.tanh(gin + r * ghn)
    return (1.0 - z) * nn_ + z * h


def _gru_body(x_ref, wrz0, brz0, wni0, bni0, wnh0, bnh0,
              wrz1, brz1, wni1, bni1, wnh1, bnh1,
              h_out, hn_out, h1_s, h2_s):
    t = pl.program_id(1)
    T = pl.num_programs(1)
    H = wnh0.shape[0]

    @pl.when(t == 0)
    def _():
        h1_s[...] = jnp.zeros_like(h1_s)
        h2_s[...] = jnp.zeros_like(h2_s)

    xt = x_ref[0]
    h1n = _gru_step(xt, h1_s[...], wrz0[...], brz0[...], wni0[...],
                    bni0[...], wnh0[...], bnh0[...], H)
    h1_s[...] = h1n
    h2n = _gru_step(h1n, h2_s[...], wrz1[...], brz1[...], wni1[...],
                    bni1[...], wnh1[...], bnh1[...], H)
    h2_s[...] = h2n

    @pl.when(t == T - 1)
    def _():
        h_out[...] = h2n
        hn_out[...] = _rownorm(h2n)


def _run_gru(x, p):
    N, T, D = x.shape
    H = p['W_hh0'].shape[1]
    Bg = _pick_block(N, [2000, 1000, 400, 200, 80, 40, 16, 8])
    G = N // Bg
    xt = jnp.swapaxes(x, 0, 1)
    f32 = jnp.float32

    def gate_pack(wi, wh, bi, bh):
        wrz = jnp.concatenate([wi.T[:, :2 * H], wh.T[:, :2 * H]], axis=0)
        brz = (bi[:2 * H] + bh[:2 * H]).reshape(1, -1)
        return (wrz, brz, wi.T[:, 2 * H:], bi[2 * H:].reshape(1, -1),
                wh.T[:, 2 * H:], bh[2 * H:].reshape(1, -1))

    l0 = gate_pack(p['W_ih0'], p['W_hh0'], p['b_ih0'], p['b_hh0'])
    l1 = gate_pack(p['W_ih1'], p['W_hh1'], p['b_ih1'], p['b_hh1'])

    wspec = pl.BlockSpec(None, lambda g, t: (0, 0))
    h_out, hn_out = pl.pallas_call(
        _gru_body,
        grid=(G, T),
        in_specs=[
            pl.BlockSpec((1, Bg, D), lambda g, t: (t, g, 0)),
        ] + [wspec] * 12,
        out_specs=[
            pl.BlockSpec((Bg, H), lambda g, t: (g, 0)),
            pl.BlockSpec((Bg, H), lambda g, t: (g, 0)),
        ],
        out_shape=[
            jax.ShapeDtypeStruct((N, H), f32),
            jax.ShapeDtypeStruct((N, H), f32),
        ],
        scratch_shapes=[
            pltpu.VMEM((Bg, H), f32),
            pltpu.VMEM((Bg, H), f32),
        ],
    )(xt, *l0, *l1)
    return h_out, hn_out


def _topk_body(hn_blk, hn_full, h_blk, pre_ref, cs_ref, *, K, B):
    i = pl.program_id(0)
    N = hn_full.shape[0]

    hnb = hn_blk[...]
    S = lax.dot_general(hnb, hn_full[...], (((1,), (1,)), ((), ())))
    rows = i * B + lax.broadcasted_iota(jnp.int32, (B, N), 0)
    cols = lax.broadcasted_iota(jnp.int32, (B, N), 1)
    S = jnp.where(rows == cols, 0.0, S)
    A = jnp.abs(S)

    def body(_, carry):
        v, _t = carry
        m = jnp.max(v, axis=1, keepdims=True)
        return jnp.where(v >= m, -1.0, v), m

    _, thr = lax.fori_loop(0, K, body, (A, jnp.zeros((B, 1), jnp.float32)))
    TS = jnp.where(A >= thr, S, 0.0)

    @pl.when(i == 0)
    def _():
        pre_ref[...] = jnp.zeros_like(pre_ref)
        cs_ref[...] = jnp.zeros_like(cs_ref)

    pre_ref[...] += lax.dot_general(TS, h_blk[...], (((0,), (0,)), ((), ())))
    cs_ref[...] += lax.dot_general(TS, jnp.ones((B, 1), jnp.float32),
                                   (((0,), (0,)), ((), ())))


def _run_topk_accum(hn, h, K):
    N, H = h.shape
    B = _pick_block(N, [200, 80, 40, 16, 8])
    f32 = jnp.float32
    pre, cs = pl.pallas_call(
        functools.partial(_topk_body, K=K, B=B),
        grid=(N // B,),
        in_specs=[
            pl.BlockSpec((B, H), lambda i: (i, 0)),
            pl.BlockSpec(None, lambda i: (0, 0)),
            pl.BlockSpec((B, H), lambda i: (i, 0)),
        ],
        out_specs=[
            pl.BlockSpec(None, lambda i: (0, 0)),
            pl.BlockSpec(None, lambda i: (0, 0)),
        ],
        out_shape=[
            jax.ShapeDtypeStruct((N, H), f32),
            jax.ShapeDtypeStruct((N, 1), f32),
        ],
    )(hn, hn, h)
    return pre, cs


def _att_body(pre_ref, cs_ref, hn_ref, h_ref, wc, bc,
              whs, bhs, wback, bback, wfore, bfore, windi, bindi,
              wout, bout, pred_ref, concept_s, cn_s, acc_s, *, Bc):
    i = pl.program_id(0)
    G = pl.num_programs(0)

    @pl.when(i == 0)
    def _():
        hn = hn_ref[...]
        n2 = jnp.sum(hn * hn, axis=1, keepdims=True)
        d = jnp.where((cs_ref[...] != 0) & (n2 > 0.5), 1.0, 0.0)
        pre_c = pre_ref[...] + d * h_ref[...]
        valid = jnp.where(jnp.sum(pre_c, axis=1, keepdims=True) != 0, 1.0, 0.0)
        concept = _leaky(jnp.dot(pre_c, wc[...]) + bc[...]) * valid
        concept_s[...] = concept
        cn_s[...] = _rownorm(concept)
        acc_s[...] = jnp.zeros_like(acc_s)

    cn_j = cn_s[pl.ds(i * Bc, Bc), :]
    C = lax.dot_general(hn_ref[...], cn_j, (((1,), (1,)), ((), ())))
    e = jnp.exp(C - 1.0)
    s = jnp.sum(e, axis=0, keepdims=True)
    att = e / s
    acc_s[...] += jnp.dot(att, concept_s[pl.ds(i * Bc, Bc), :])

    @pl.when(i == G - 1)
    def _():
        hsh = acc_s[...]
        hs = _leaky(jnp.dot(hsh, whs[...]) + bhs[...])
        hb = _leaky(jnp.dot(hs, wback[...]) + bback[...])
        ofs = _leaky(jnp.dot(hs, wfore[...]) + bfore[...])
        indi = h_ref[...] - hb
        oin = _leaky(jnp.dot(indi, windi[...]) + bindi[...])
        pred_ref[...] = jnp.dot(ofs + oin, wout[...]) + bout[...]


def _run_att_head(pre, cs, hn, h, p):
    N, H = h.shape
    Bc = _pick_block(N, [400, 200, 80, 40, 16, 8])
    f32 = jnp.float32
    spec = pl.BlockSpec(None, lambda i: (0, 0))
    pred = pl.pallas_call(
        functools.partial(_att_body, Bc=Bc),
        grid=(N // Bc,),
        in_specs=[spec] * 16,
        out_specs=pl.BlockSpec(None, lambda i: (0, 0)),
        out_shape=jax.ShapeDtypeStruct((N, 1), f32),
        scratch_shapes=[
            pltpu.VMEM((N, H), f32),
            pltpu.VMEM((N, H), f32),
            pltpu.VMEM((N, H), f32),
        ],
    )(
        pre, cs, hn, h,
        p['W_c'].T, p['b_c'].reshape(1, -1),
        p['W_hs'].T, p['b_hs'].reshape(1, -1),
        p['W_back'].T, p['b_back'].reshape(1, -1),
        p['W_fore'].T, p['b_fore'].reshape(1, -1),
        p['W_indi'].T, p['b_indi'].reshape(1, -1),
        p['W_out'].T, p['b_out'].reshape(1, -1),
    )
    return pred


K_TOP = 20


def kernel(x, params):
    h, hn = _run_gru(x, params)
    pre, cs = _run_topk_accum(hn, h, K_TOP)
    pred = _run_att_head(pre, cs, hn, h, params)
    return pred[:, 0]


# speedup vs baseline: 7.9712x; 1.0904x over previous
import functools

import jax
import jax.numpy as jnp
from jax import lax
from jax.experimental import pallas as pl
from jax.experimental.pallas import tpu as pltpu


def _pick_block(n, candidates):
    for c in candidates:
        if n % c == 0:
            return c
    return n


def _leaky(x):
    return jnp.where(x >= 0, x, 0.2 * x)


def _rownorm(h):
    n2 = jnp.sum(h * h, axis=1, keepdims=True)
    inv = lax.rsqrt(jnp.where(n2 > 0, n2, 1.0))
    return jnp.where(n2 > 0, h * inv, 0.0)


def _gru_step(xt, h, wrz, brz, wni, bni, wnh, bnh, H):
    cat = jnp.concatenate([xt, h], axis=1)
    rz = jax.nn.sigmoid(jnp.dot(cat, wrz) + brz)
    r = rz[:, :H]
    z = rz[:, H:]
    gin = jnp.dot(xt, wni) + bni
    ghn = jnp.dot(h, wnh) + bnh
    nn_ = jnp.tanh(gin + r * ghn)
    return (1.0 - z) * nn_ + z * h


def _gru_body(x_ref, wrz0, brz0, wni0, bni0, wnh0, bnh0,
              wrz1, brz1, wni1, bni1, wnh1, bnh1,
              h_out, hn_out, h1_s, h2_s):
    t = pl.program_id(1)
    T = pl.num_programs(1)
    H = wnh0.shape[0]

    @pl.when(t == 0)
    def _():
        h1_s[...] = jnp.zeros_like(h1_s)
        h2_s[...] = jnp.zeros_like(h2_s)

    xt = x_ref[0]
    h1n = _gru_step(xt, h1_s[...], wrz0[...], brz0[...], wni0[...],
                    bni0[...], wnh0[...], bnh0[...], H)
    h1_s[...] = h1n
    h2n = _gru_step(h1n, h2_s[...], wrz1[...], brz1[...], wni1[...],
                    bni1[...], wnh1[...], bnh1[...], H)
    h2_s[...] = h2n

    @pl.when(t == T - 1)
    def _():
        h_out[...] = h2n
        hn_out[...] = _rownorm(h2n)


def _run_gru(x, p):
    N, T, D = x.shape
    H = p['W_hh0'].shape[1]
    Bg = _pick_block(N, [2000, 1000, 400, 200, 80, 40, 16, 8])
    G = N // Bg
    xt = jnp.swapaxes(x, 0, 1)
    f32 = jnp.float32

    def gate_pack(wi, wh, bi, bh):
        wrz = jnp.concatenate([wi.T[:, :2 * H], wh.T[:, :2 * H]], axis=0)
        brz = (bi[:2 * H] + bh[:2 * H]).reshape(1, -1)
        return (wrz, brz, wi.T[:, 2 * H:], bi[2 * H:].reshape(1, -1),
                wh.T[:, 2 * H:], bh[2 * H:].reshape(1, -1))

    l0 = gate_pack(p['W_ih0'], p['W_hh0'], p['b_ih0'], p['b_hh0'])
    l1 = gate_pack(p['W_ih1'], p['W_hh1'], p['b_ih1'], p['b_hh1'])

    wspec = pl.BlockSpec(None, lambda g, t: (0, 0))
    h_out, hn_out = pl.pallas_call(
        _gru_body,
        grid=(G, T),
        in_specs=[
            pl.BlockSpec((1, Bg, D), lambda g, t: (t, g, 0)),
        ] + [wspec] * 12,
        out_specs=[
            pl.BlockSpec((Bg, H), lambda g, t: (g, 0)),
            pl.BlockSpec((Bg, H), lambda g, t: (g, 0)),
        ],
        out_shape=[
            jax.ShapeDtypeStruct((N, H), f32),
            jax.ShapeDtypeStruct((N, H), f32),
        ],
        scratch_shapes=[
            pltpu.VMEM((Bg, H), f32),
            pltpu.VMEM((Bg, H), f32),
        ],
    )(xt, *l0, *l1)
    return h_out, hn_out


def _topk_body(hn_blk, hn_full, h_blk, pre_ref, cs_ref, *, K, B, N):
    i = pl.program_id(0)
    NP = hn_full.shape[0]

    hnb = hn_blk[...]
    S = lax.dot_general(hnb, hn_full[...], (((1,), (1,)), ((), ())))
    rows = i * B + lax.broadcasted_iota(jnp.int32, (B, NP), 0)
    cols = lax.broadcasted_iota(jnp.int32, (B, NP), 1)
    S = jnp.where(rows == cols, 0.0, S)
    A = jnp.abs(S)
    A = jnp.where(cols >= N, -1.0, A)

    Q = NP // 4
    a, b = A[:, 0 * Q:1 * Q], A[:, 1 * Q:2 * Q]
    c, d = A[:, 2 * Q:3 * Q], A[:, 3 * Q:4 * Q]
    ab_hi, ab_lo = jnp.maximum(a, b), jnp.minimum(a, b)
    cd_hi, cd_lo = jnp.maximum(c, d), jnp.minimum(c, d)
    q1 = jnp.maximum(ab_hi, cd_hi)
    t_ = jnp.minimum(ab_hi, cd_hi)
    q4 = jnp.minimum(ab_lo, cd_lo)
    u_ = jnp.maximum(ab_lo, cd_lo)
    q2 = jnp.maximum(t_, u_)
    q3 = jnp.minimum(t_, u_)

    def body(_, carry):
        w1, w2, w3, w4, _t = carry
        m = jnp.max(w1, axis=1, keepdims=True)
        sel = w1 >= m
        return (jnp.where(sel, w2, w1), jnp.where(sel, w3, w2),
                jnp.where(sel, w4, w3), jnp.where(sel, -1.0, w4), m)

    thr0 = jnp.zeros((B, 1), jnp.float32)
    *_, thr = lax.fori_loop(0, K, body, (q1, q2, q3, q4, thr0))
    TS = jnp.where(A >= thr, S, 0.0)

    @pl.when(i == 0)
    def _():
        pre_ref[...] = jnp.zeros_like(pre_ref)
        cs_ref[...] = jnp.zeros_like(cs_ref)

    pre_ref[...] += lax.dot_general(TS, h_blk[...], (((0,), (0,)), ((), ())))
    cs_ref[...] += lax.dot_general(TS, jnp.ones((B, 1), jnp.float32),
                                   (((0,), (0,)), ((), ())))


def _round_up(n, m):
    return ((n + m - 1) // m) * m


def _run_topk_accum(hn, h, K):
    N, H = h.shape
    B = _pick_block(N, [200, 80, 40, 16, 8])
    NP = _round_up(N, 4096) if N >= 4096 else _round_up(N, 64)
    hnp = jnp.pad(hn, ((0, NP - N), (0, 0)))
    f32 = jnp.float32
    pre, cs = pl.pallas_call(
        functools.partial(_topk_body, K=K, B=B, N=N),
        grid=(N // B,),
        in_specs=[
            pl.BlockSpec((B, H), lambda i: (i, 0)),
            pl.BlockSpec(None, lambda i: (0, 0)),
            pl.BlockSpec((B, H), lambda i: (i, 0)),
        ],
        out_specs=[
            pl.BlockSpec(None, lambda i: (0, 0)),
            pl.BlockSpec(None, lambda i: (0, 0)),
        ],
        out_shape=[
            jax.ShapeDtypeStruct((NP, H), f32),
            jax.ShapeDtypeStruct((NP, 1), f32),
        ],
    )(hn, hnp, h)
    return pre[:N], cs[:N]


def _att_body(pre_ref, cs_ref, hn_ref, h_ref, wc, bc,
              whs, bhs, wback, bback, wfore, bfore, windi, bindi,
              wout, bout, pred_ref, concept_s, cn_s, acc_s, *, Bc):
    i = pl.program_id(0)
    G = pl.num_programs(0)

    @pl.when(i == 0)
    def _():
        hn = hn_ref[...]
        n2 = jnp.sum(hn * hn, axis=1, keepdims=True)
        d = jnp.where((cs_ref[...] != 0) & (n2 > 0.5), 1.0, 0.0)
        pre_c = pre_ref[...] + d * h_ref[...]
        valid = jnp.where(jnp.sum(pre_c, axis=1, keepdims=True) != 0, 1.0, 0.0)
        concept = _leaky(jnp.dot(pre_c, wc[...]) + bc[...]) * valid
        concept_s[...] = concept
        cn_s[...] = _rownorm(concept)
        acc_s[...] = jnp.zeros_like(acc_s)

    cn_j = cn_s[pl.ds(i * Bc, Bc), :]
    C = lax.dot_general(hn_ref[...], cn_j, (((1,), (1,)), ((), ())))
    e = jnp.exp(C - 1.0)
    s = jnp.sum(e, axis=0, keepdims=True)
    att = e / s
    acc_s[...] += jnp.dot(att, concept_s[pl.ds(i * Bc, Bc), :])

    @pl.when(i == G - 1)
    def _():
        hsh = acc_s[...]
        hs = _leaky(jnp.dot(hsh, whs[...]) + bhs[...])
        hb = _leaky(jnp.dot(hs, wback[...]) + bback[...])
        ofs = _leaky(jnp.dot(hs, wfore[...]) + bfore[...])
        indi = h_ref[...] - hb
        oin = _leaky(jnp.dot(indi, windi[...]) + bindi[...])
        pred_ref[...] = jnp.dot(ofs + oin, wout[...]) + bout[...]


def _run_att_head(pre, cs, hn, h, p):
    N, H = h.shape
    Bc = _pick_block(N, [400, 200, 80, 40, 16, 8])
    f32 = jnp.float32
    spec = pl.BlockSpec(None, lambda i: (0, 0))
    pred = pl.pallas_call(
        functools.partial(_att_body, Bc=Bc),
        grid=(N // Bc,),
        in_specs=[spec] * 16,
        out_specs=pl.BlockSpec(None, lambda i: (0, 0)),
        out_shape=jax.ShapeDtypeStruct((N, 1), f32),
        scratch_shapes=[
            pltpu.VMEM((N, H), f32),
            pltpu.VMEM((N, H), f32),
            pltpu.VMEM((N, H), f32),
        ],
    )(
        pre, cs, hn, h,
        p['W_c'].T, p['b_c'].reshape(1, -1),
        p['W_hs'].T, p['b_hs'].reshape(1, -1),
        p['W_back'].T, p['b_back'].reshape(1, -1),
        p['W_fore'].T, p['b_fore'].reshape(1, -1),
        p['W_indi'].T, p['b_indi'].reshape(1, -1),
        p['W_out'].T, p['b_out'].reshape(1, -1),
    )
    return pred


K_TOP = 20


def kernel(x, params):
    h, hn = _run_gru(x, params)
    pre, cs = _run_topk_accum(hn, h, K_TOP)
    pred = _run_att_head(pre, cs, hn, h, params)
    return pred[:, 0]
